# baseline (speedup 1.0000x reference)
import functools

import jax
import jax.numpy as jnp
from jax.experimental import pallas as pl
from jax.experimental.pallas import tpu as pltpu


def _fused_kernel(nk, xT_ref, A_ref, U0_ref, b0_ref, U1_ref, U2_ref, out_ref,
                  acc_ref, deg_ref, abf_ref, z2_ref):
    l = pl.program_id(0)
    k = pl.program_id(1)
    bk = abf_ref.shape[1]

    @pl.when(k == 0)
    def _reset_acc():
        acc_ref[...] = jnp.zeros_like(acc_ref)

    @pl.when(jnp.logical_and(l == 0, k == 0))
    def _reset_deg():
        deg_ref[...] = jnp.zeros_like(deg_ref)

    @pl.when(l == 0)
    def _layer1_step():
        A_raw = A_ref[...]
        Af = A_raw.astype(jnp.bfloat16)
        abf_ref[k] = Af
        y0 = jax.lax.dot_general(
            U0_ref[...], xT_ref[...], (((0,), (0,)), ((), ())),
            preferred_element_type=jnp.float32) + b0_ref[...]
        z1 = jnp.dot(U1_ref[...], y0, preferred_element_type=jnp.float32)
        acc_ref[...] += jnp.dot(z1.astype(jnp.bfloat16), Af,
                                preferred_element_type=jnp.float32)
        deg_ref[...] += jnp.sum(A_raw, axis=0, keepdims=True).astype(jnp.float32)

        @pl.when(k == nk - 1)
        def _layer1_out():
            y1 = jnp.maximum(acc_ref[...] * (1.0 / deg_ref[...]), 0.0)
            z2_ref[...] = jnp.dot(U2_ref[...], y1,
                                  preferred_element_type=jnp.float32
                                  ).astype(jnp.bfloat16)

    @pl.when(l == 1)
    def _layer2_step():
        acc_ref[...] += jnp.dot(z2_ref[:, pl.ds(k * bk, bk)], abf_ref[k],
                                preferred_element_type=jnp.float32)

        @pl.when(k == nk - 1)
        def _layer2_out():
            out_ref[...] = jnp.maximum(acc_ref[...] * (1.0 / deg_ref[...]), 0.0)


def kernel(x, adj_mat, U0, b0, U1, U2):
    N, D = x.shape
    bk = 256
    nk = N // bk
    xT = x.T
    b0c = b0.reshape(D, 1)
    yT = pl.pallas_call(
        functools.partial(_fused_kernel, nk),
        grid=(2, nk),
        in_specs=[
            pl.BlockSpec((D, bk),
                         lambda l, k: (0, jnp.where(l == 0, k, nk - 1))),
            pl.BlockSpec((bk, N),
                         lambda l, k: (jnp.where(l == 0, k, nk - 1), 0)),
            pl.BlockSpec((D, D), lambda l, k: (0, 0)),
            pl.BlockSpec((D, 1), lambda l, k: (0, 0)),
            pl.BlockSpec((D, D), lambda l, k: (0, 0)),
            pl.BlockSpec((D, D), lambda l, k: (0, 0)),
        ],
        out_specs=pl.BlockSpec((D, N), lambda l, k: (0, 0)),
        out_shape=jax.ShapeDtypeStruct((D, N), jnp.float32),
        scratch_shapes=[
            pltpu.VMEM((D, N), jnp.float32),
            pltpu.VMEM((1, N), jnp.float32),
            pltpu.VMEM((nk, bk, N), jnp.bfloat16),
            pltpu.VMEM((D, N), jnp.bfloat16),
        ],
        compiler_params=pltpu.CompilerParams(
            dimension_semantics=("arbitrary", "arbitrary")),
    )(xT, adj_mat, U0, b0c, U1, U2)
    return yT.T


# speedup vs baseline: 1.2172x; 1.2172x over previous
import functools

import jax
import jax.numpy as jnp
from jax.experimental import pallas as pl
from jax.experimental.pallas import tpu as pltpu


def _fused_kernel(nk, xT_ref, A_ref, U0_ref, b0_ref, U1_ref, U2_ref, out_ref,
                  acc_ref, deg_ref, abf_ref, z2_ref, w1_ref, c1_ref):
    l = pl.program_id(0)
    k = pl.program_id(1)
    bk = abf_ref.shape[1]

    @pl.when(k == 0)
    def _reset_acc():
        acc_ref[...] = jnp.zeros_like(acc_ref)

    @pl.when(jnp.logical_and(l == 0, k == 0))
    def _prologue():
        deg_ref[...] = jnp.zeros_like(deg_ref)
        w1_ref[...] = jax.lax.dot_general(
            U1_ref[...], U0_ref[...], (((1,), (1,)), ((), ())),
            preferred_element_type=jnp.float32).astype(jnp.bfloat16)
        c1_ref[...] = jnp.dot(U1_ref[...], b0_ref[...],
                              preferred_element_type=jnp.float32)

    @pl.when(l == 0)
    def _layer1_step():
        A_raw = A_ref[...]
        Af = A_raw.astype(jnp.bfloat16)
        abf_ref[k] = Af
        z1 = jnp.dot(w1_ref[...], xT_ref[...].astype(jnp.bfloat16),
                     preferred_element_type=jnp.float32) + c1_ref[...]
        acc_ref[...] += jnp.dot(z1.astype(jnp.bfloat16), Af,
                                preferred_element_type=jnp.float32)
        deg_ref[...] += jnp.sum(A_raw, axis=0, keepdims=True).astype(jnp.float32)

        @pl.when(k == nk - 1)
        def _layer1_out():
            y1 = jnp.maximum(acc_ref[...] * (1.0 / deg_ref[...]), 0.0)
            z2_ref[...] = jnp.dot(U2_ref[...].astype(jnp.bfloat16),
                                  y1.astype(jnp.bfloat16),
                                  preferred_element_type=jnp.float32
                                  ).astype(jnp.bfloat16)

    @pl.when(l == 1)
    def _layer2_step():
        acc_ref[...] += jnp.dot(z2_ref[:, pl.ds(k * bk, bk)], abf_ref[k],
                                preferred_element_type=jnp.float32)

        @pl.when(k == nk - 1)
        def _layer2_out():
            out_ref[...] = jnp.maximum(acc_ref[...] * (1.0 / deg_ref[...]), 0.0)


def kernel(x, adj_mat, U0, b0, U1, U2):
    N, D = x.shape
    bk = 512
    nk = N // bk
    xT = x.T
    b0c = b0.reshape(D, 1)
    yT = pl.pallas_call(
        functools.partial(_fused_kernel, nk),
        grid=(2, nk),
        in_specs=[
            pl.BlockSpec((D, bk),
                         lambda l, k: (0, jnp.where(l == 0, k, nk - 1))),
            pl.BlockSpec((bk, N),
                         lambda l, k: (jnp.where(l == 0, k, nk - 1), 0)),
            pl.BlockSpec((D, D), lambda l, k: (0, 0)),
            pl.BlockSpec((D, 1), lambda l, k: (0, 0)),
            pl.BlockSpec((D, D), lambda l, k: (0, 0)),
            pl.BlockSpec((D, D), lambda l, k: (0, 0)),
        ],
        out_specs=pl.BlockSpec((D, N), lambda l, k: (0, 0)),
        out_shape=jax.ShapeDtypeStruct((D, N), jnp.float32),
        scratch_shapes=[
            pltpu.VMEM((D, N), jnp.float32),
            pltpu.VMEM((1, N), jnp.float32),
            pltpu.VMEM((nk, bk, N), jnp.bfloat16),
            pltpu.VMEM((D, N), jnp.bfloat16),
            pltpu.VMEM((D, D), jnp.bfloat16),
            pltpu.VMEM((D, 1), jnp.float32),
        ],
        compiler_params=pltpu.CompilerParams(
            dimension_semantics=("arbitrary", "arbitrary")),
    )(xT, adj_mat, U0, b0c, U1, U2)
    return yT.T


# speedup vs baseline: 1.4268x; 1.1721x over previous
import functools

import jax
import jax.numpy as jnp
from jax.experimental import pallas as pl
from jax.experimental.pallas import tpu as pltpu


def _fused_kernel(nk, x_ref, A_ref, U0_ref, b0_ref, U1_ref, U2_ref, out_ref,
                  acc_ref, deg_ref, abf_ref, z2_ref, w1_ref, c1_ref):
    l = pl.program_id(0)
    k = pl.program_id(1)
    bk = abf_ref.shape[1]
    D = acc_ref.shape[0]

    @pl.when(jnp.logical_and(l == 0, k == 0))
    def _prologue():
        acc_ref[...] = jnp.zeros_like(acc_ref)
        deg_ref[...] = jnp.zeros_like(deg_ref)
        w1_ref[...] = jax.lax.dot_general(
            U1_ref[...], U0_ref[...], (((1,), (1,)), ((), ())),
            preferred_element_type=jnp.float32).astype(jnp.bfloat16)
        c1_ref[...] = jnp.dot(U1_ref[...], b0_ref[...],
                              preferred_element_type=jnp.float32)

    @pl.when(l == 0)
    def _layer1_step():
        A_raw = A_ref[...]
        Af = A_raw.astype(jnp.bfloat16)
        abf_ref[k] = Af
        z1 = jax.lax.dot_general(
            w1_ref[...], x_ref[...].astype(jnp.bfloat16),
            (((1,), (1,)), ((), ())),
            preferred_element_type=jnp.float32) + c1_ref[...]
        acc_ref[...] += jnp.dot(z1.astype(jnp.bfloat16), Af,
                                preferred_element_type=jnp.float32)
        deg_ref[...] += jnp.sum(A_raw, axis=0, keepdims=True).astype(jnp.float32)

        @pl.when(k == nk - 1)
        def _layer1_out():
            inv = 1.0 / deg_ref[...]
            deg_ref[...] = inv
            y1 = jnp.maximum(acc_ref[...] * inv, 0.0)
            z2_ref[...] = jnp.dot(U2_ref[...].astype(jnp.bfloat16),
                                  y1.astype(jnp.bfloat16),
                                  preferred_element_type=jnp.float32
                                  ).astype(jnp.bfloat16)

    @pl.when(l == 1)
    def _layer2_step():
        acc2 = jnp.zeros((D, bk), jnp.float32)
        for j in range(nk):
            acc2 += jnp.dot(z2_ref[:, j * bk:(j + 1) * bk],
                            abf_ref[j, :, pl.ds(k * bk, bk)],
                            preferred_element_type=jnp.float32)
        aggT = jnp.maximum(acc2 * deg_ref[:, pl.ds(k * bk, bk)], 0.0)
        out_ref[...] = aggT.T


def kernel(x, adj_mat, U0, b0, U1, U2):
    N, D = x.shape
    bk = 512
    nk = N // bk
    b0c = b0.reshape(D, 1)
    return pl.pallas_call(
        functools.partial(_fused_kernel, nk),
        grid=(2, nk),
        in_specs=[
            pl.BlockSpec((bk, D),
                         lambda l, k: (jnp.where(l == 0, k, nk - 1), 0)),
            pl.BlockSpec((bk, N),
                         lambda l, k: (jnp.where(l == 0, k, nk - 1), 0)),
            pl.BlockSpec((D, D), lambda l, k: (0, 0)),
            pl.BlockSpec((D, 1), lambda l, k: (0, 0)),
            pl.BlockSpec((D, D), lambda l, k: (0, 0)),
            pl.BlockSpec((D, D), lambda l, k: (0, 0)),
        ],
        out_specs=pl.BlockSpec((bk, D),
                               lambda l, k: (jnp.where(l == 0, 0, k), 0)),
        out_shape=jax.ShapeDtypeStruct((N, D), jnp.float32),
        scratch_shapes=[
            pltpu.VMEM((D, N), jnp.float32),
            pltpu.VMEM((1, N), jnp.float32),
            pltpu.VMEM((nk, bk, N), jnp.bfloat16),
            pltpu.VMEM((D, N), jnp.bfloat16),
            pltpu.VMEM((D, D), jnp.bfloat16),
            pltpu.VMEM((D, 1), jnp.float32),
        ],
        compiler_params=pltpu.CompilerParams(
            dimension_semantics=("arbitrary", "arbitrary")),
    )(x, adj_mat, U0, b0c, U1, U2)
